# Initial kernel scaffold; baseline (speedup 1.0000x reference)
import functools

import jax
import jax.numpy as jnp
from jax import lax
from jax.experimental import pallas as pl
from jax.experimental.pallas import tpu as pltpu
from jax.experimental.pallas import tpu_sc as plsc

NUM_CORES = 2
NUM_SUBCORES = 16
NUM_TILES = NUM_CORES * NUM_SUBCORES
EDGE_CHUNK = 80


def _sc_mesh():
    return plsc.VectorSubcoreMesh(
        core_axis_name="c", subcore_axis_name="s",
        num_cores=NUM_CORES, num_subcores=NUM_SUBCORES)


def _make_degree_kernel(n_nodes, n_edges):
    epw = n_edges // NUM_TILES
    nchunk = epw // EDGE_CHUNK
    rows_per = n_nodes // NUM_SUBCORES

    @functools.partial(
        pl.kernel,
        out_type=jax.ShapeDtypeStruct((NUM_CORES, n_nodes, 16), jnp.float32),
        mesh=_sc_mesh(),
        scratch_types=[
            pltpu.VMEM((EDGE_CHUNK,), jnp.int32),
            pltpu.VMEM((EDGE_CHUNK, 16), jnp.float32),
            pltpu.VMEM_SHARED((n_nodes, 16), jnp.float32),
        ],
    )
    def deg_kernel(dst_hbm, zeros_hbm, ones_hbm, out_hbm, didx, ones_v, acc):
        c = lax.axis_index("c")
        s = lax.axis_index("s")
        w = c * NUM_SUBCORES + s
        pltpu.sync_copy(zeros_hbm.at[pl.ds(s * rows_per, rows_per)],
                        acc.at[pl.ds(s * rows_per, rows_per)])
        pltpu.sync_copy(ones_hbm, ones_v)
        plsc.subcore_barrier()

        def body(j, carry):
            base = w * epw + j * EDGE_CHUNK
            pltpu.sync_copy(dst_hbm.at[pl.ds(base, EDGE_CHUNK)], didx)
            pltpu.sync_copy(ones_v, acc.at[didx], add=True)
            return carry

        lax.fori_loop(0, nchunk, body, 0)
        plsc.subcore_barrier()
        pltpu.sync_copy(acc.at[pl.ds(s * rows_per, rows_per)],
                        out_hbm.at[c, pl.ds(s * rows_per, rows_per)])

    return deg_kernel


def _make_agg_kernel(n_nodes, n_edges, hdim):
    epw = n_edges // NUM_TILES
    nchunk = epw // EDGE_CHUNK
    rows_per = n_nodes // NUM_SUBCORES

    @functools.partial(
        pl.kernel,
        out_type=jax.ShapeDtypeStruct((NUM_CORES, n_nodes, hdim), jnp.float32),
        mesh=_sc_mesh(),
        scratch_types=[
            pltpu.VMEM((EDGE_CHUNK,), jnp.int32),
            pltpu.VMEM((EDGE_CHUNK,), jnp.int32),
            pltpu.VMEM((EDGE_CHUNK, hdim), jnp.float32),
            pltpu.VMEM_SHARED((n_nodes, hdim), jnp.float32),
            pltpu.SemaphoreType.DMA,
        ],
    )
    def agg_kernel(hs_hbm, src_hbm, dst_hbm, out_hbm, sidx, didx, rows, acc, sem):
        c = lax.axis_index("c")
        s = lax.axis_index("s")
        w = c * NUM_SUBCORES + s
        pltpu.sync_copy(hs_hbm.at[pl.ds(s * rows_per, rows_per)],
                        acc.at[pl.ds(s * rows_per, rows_per)])
        plsc.subcore_barrier()

        def body(j, carry):
            base = w * epw + j * EDGE_CHUNK
            pltpu.sync_copy(src_hbm.at[pl.ds(base, EDGE_CHUNK)], sidx)
            pltpu.sync_copy(dst_hbm.at[pl.ds(base, EDGE_CHUNK)], didx)
            pltpu.async_copy(hs_hbm.at[sidx], rows, sem).wait()
            pltpu.sync_copy(rows, acc.at[didx], add=True)
            return carry

        lax.fori_loop(0, nchunk, body, 0)
        plsc.subcore_barrier()
        pltpu.sync_copy(acc.at[pl.ds(s * rows_per, rows_per)],
                        out_hbm.at[c, pl.ds(s * rows_per, rows_per)])

    return agg_kernel


ROW_BLK = 1000


def _tc1_body(dp_ref, x_ref, w1_ref, dis_ref, h1s_ref):
    deg = dp_ref[0] + dp_ref[1] + 1.0
    dis = lax.rsqrt(deg)
    dis_ref[...] = dis
    h = jnp.dot(x_ref[...], w1_ref[...], preferred_element_type=jnp.float32)
    h1s_ref[...] = h * dis[:, :1]


def _tc2_body(p_ref, h1s_ref, dis_ref, b1_ref, w2_ref, h2s_ref):
    dis = dis_ref[...][:, :1]
    agg = p_ref[0] + p_ref[1] - h1s_ref[...]
    a1 = jnp.maximum(agg * dis + b1_ref[...], 0.0)
    h2 = jnp.dot(a1, w2_ref[...], preferred_element_type=jnp.float32)
    h2s_ref[...] = h2 * dis


def _tc3_body(q_ref, h2s_ref, dis_ref, b2_ref, batch_ref, wout_ref, bout_ref,
              out_ref, sums, cnts, *, nblk, n_graphs):
    i = pl.program_id(0)

    @pl.when(i == 0)
    def _init():
        sums[...] = jnp.zeros_like(sums)
        cnts[...] = jnp.zeros_like(cnts)

    dis = dis_ref[...][:, :1]
    agg = q_ref[0] + q_ref[1] - h2s_ref[...]
    a2 = jnp.maximum(agg * dis + b2_ref[...], 0.0)

    seg = batch_ref[0, 0, :]
    iota = lax.broadcasted_iota(jnp.int32, (a2.shape[0], n_graphs), 1)
    onehot = (seg[:, None] == iota).astype(jnp.float32)
    dims = (((0,), (0,)), ((), ()))
    sums[...] += lax.dot_general(onehot, a2, dims,
                                 preferred_element_type=jnp.float32)
    cnts[...] += lax.dot_general(onehot, jnp.ones_like(a2), dims,
                                 preferred_element_type=jnp.float32)

    @pl.when(i == nblk - 1)
    def _fin():
        g = sums[...] / jnp.maximum(cnts[...], 1.0)
        logits = (jnp.dot(g, wout_ref[...], preferred_element_type=jnp.float32)
                  + bout_ref[...])
        m = jnp.max(logits, axis=1, keepdims=True)
        z = logits - m
        lse = jnp.log(jnp.sum(jnp.exp(z), axis=1, keepdims=True))
        out_ref[...] = z - lse


def kernel(x, edge_index, batch, W1, b1, W2, b2, Wout, bout):
    n, f_in = x.shape
    h = W1.shape[1]
    g_graphs = 64
    c_out = Wout.shape[1]
    e = edge_index.shape[1]
    src = edge_index[0]
    dst = edge_index[1]

    nblk = n // ROW_BLK

    deg_kernel = _make_degree_kernel(n, e)
    zeros16 = jnp.zeros((n, 16), jnp.float32)
    ones16 = jnp.ones((EDGE_CHUNK, 16), jnp.float32)
    degp = deg_kernel(dst, zeros16, ones16)

    dis, h1s = pl.pallas_call(
        _tc1_body,
        grid=(nblk,),
        in_specs=[
            pl.BlockSpec((NUM_CORES, ROW_BLK, 16), lambda i: (0, i, 0)),
            pl.BlockSpec((ROW_BLK, f_in), lambda i: (i, 0)),
            pl.BlockSpec((f_in, h), lambda i: (0, 0)),
        ],
        out_specs=[
            pl.BlockSpec((ROW_BLK, 16), lambda i: (i, 0)),
            pl.BlockSpec((ROW_BLK, h), lambda i: (i, 0)),
        ],
        out_shape=[
            jax.ShapeDtypeStruct((n, 16), jnp.float32),
            jax.ShapeDtypeStruct((n, h), jnp.float32),
        ],
    )(degp, x, W1)

    agg_kernel = _make_agg_kernel(n, e, h)
    p = agg_kernel(h1s, src, dst)

    h2s = pl.pallas_call(
        _tc2_body,
        grid=(nblk,),
        in_specs=[
            pl.BlockSpec((NUM_CORES, ROW_BLK, h), lambda i: (0, i, 0)),
            pl.BlockSpec((ROW_BLK, h), lambda i: (i, 0)),
            pl.BlockSpec((ROW_BLK, 16), lambda i: (i, 0)),
            pl.BlockSpec((1, h), lambda i: (0, 0)),
            pl.BlockSpec((h, h), lambda i: (0, 0)),
        ],
        out_specs=pl.BlockSpec((ROW_BLK, h), lambda i: (i, 0)),
        out_shape=jax.ShapeDtypeStruct((n, h), jnp.float32),
    )(p, h1s, dis, b1.reshape(1, h), W2)

    q = agg_kernel(h2s, src, dst)

    batch3d = batch.reshape(nblk, 1, ROW_BLK)
    out = pl.pallas_call(
        functools.partial(_tc3_body, nblk=nblk, n_graphs=g_graphs),
        grid=(nblk,),
        in_specs=[
            pl.BlockSpec((NUM_CORES, ROW_BLK, h), lambda i: (0, i, 0)),
            pl.BlockSpec((ROW_BLK, h), lambda i: (i, 0)),
            pl.BlockSpec((ROW_BLK, 16), lambda i: (i, 0)),
            pl.BlockSpec((1, h), lambda i: (0, 0)),
            pl.BlockSpec((1, 1, ROW_BLK), lambda i: (i, 0, 0)),
            pl.BlockSpec((h, c_out), lambda i: (0, 0)),
            pl.BlockSpec((1, c_out), lambda i: (0, 0)),
        ],
        out_specs=pl.BlockSpec((g_graphs, c_out), lambda i: (0, 0)),
        out_shape=jax.ShapeDtypeStruct((g_graphs, c_out), jnp.float32),
        scratch_shapes=[
            pltpu.VMEM((g_graphs, h), jnp.float32),
            pltpu.VMEM((g_graphs, h), jnp.float32),
        ],
    )(q, h2s, dis, b2.reshape(1, h), batch3d, Wout, bout.reshape(1, c_out))

    return out


# speedup vs baseline: 12.9525x; 12.9525x over previous
import functools

import jax
import jax.numpy as jnp
from jax import lax
from jax.experimental import pallas as pl
from jax.experimental.pallas import tpu as pltpu
from jax.experimental.pallas import tpu_sc as plsc

NUM_CORES = 2
NUM_SUBCORES = 16
NUM_TILES = NUM_CORES * NUM_SUBCORES
EDGE_CHUNK = 80


def _sc_mesh():
    return plsc.VectorSubcoreMesh(
        core_axis_name="c", subcore_axis_name="s",
        num_cores=NUM_CORES, num_subcores=NUM_SUBCORES)


def _row_partition(n_nodes):
    main = (n_nodes // (NUM_SUBCORES * 8)) * 8
    tail = n_nodes - NUM_SUBCORES * main
    return main, tail


def _subcore_row_copy(src_at, dst_at, s, n_nodes):
    main, tail = _row_partition(n_nodes)
    pltpu.sync_copy(src_at(s * main, main), dst_at(s * main, main))
    if tail:
        @pl.when(s == NUM_SUBCORES - 1)
        def _tail():
            pltpu.sync_copy(src_at(NUM_SUBCORES * main, tail),
                            dst_at(NUM_SUBCORES * main, tail))


def _make_degree_kernel(n_nodes, n_edges):
    epw = n_edges // NUM_TILES
    nchunk = epw // EDGE_CHUNK

    @functools.partial(
        pl.kernel,
        out_type=jax.ShapeDtypeStruct((NUM_CORES, n_nodes, 16), jnp.float32),
        mesh=_sc_mesh(),
        scratch_types=[
            pltpu.VMEM((EDGE_CHUNK,), jnp.int32),
            pltpu.VMEM((EDGE_CHUNK, 16), jnp.float32),
            pltpu.VMEM_SHARED((n_nodes, 16), jnp.float32),
        ],
    )
    def deg_kernel(dst_hbm, zeros_hbm, ones_hbm, out_hbm, didx, ones_v, acc):
        c = lax.axis_index("c")
        s = lax.axis_index("s")
        w = c * NUM_SUBCORES + s
        _subcore_row_copy(lambda o, m: zeros_hbm.at[pl.ds(o, m)],
                          lambda o, m: acc.at[pl.ds(o, m)], s, n_nodes)
        pltpu.sync_copy(ones_hbm, ones_v)
        plsc.subcore_barrier()

        def body(j, carry):
            base = w * epw + j * EDGE_CHUNK
            pltpu.sync_copy(dst_hbm.at[pl.ds(base, EDGE_CHUNK)], didx)
            pltpu.sync_copy(ones_v, acc.at[didx], add=True)
            return carry

        lax.fori_loop(0, nchunk, body, 0)
        plsc.subcore_barrier()
        _subcore_row_copy(lambda o, m: acc.at[pl.ds(o, m)],
                          lambda o, m: out_hbm.at[c, pl.ds(o, m)], s, n_nodes)

    return deg_kernel


def _make_agg_kernel(n_nodes, n_edges, hdim):
    epw = n_edges // NUM_TILES
    nchunk = epw // EDGE_CHUNK

    @functools.partial(
        pl.kernel,
        out_type=jax.ShapeDtypeStruct((NUM_CORES, n_nodes, hdim), jnp.float32),
        mesh=_sc_mesh(),
        scratch_types=[
            pltpu.VMEM((EDGE_CHUNK,), jnp.int32),
            pltpu.VMEM((EDGE_CHUNK,), jnp.int32),
            pltpu.VMEM((EDGE_CHUNK, hdim), jnp.float32),
            pltpu.VMEM_SHARED((n_nodes, hdim), jnp.float32),
            pltpu.SemaphoreType.DMA,
        ],
    )
    def agg_kernel(hs_hbm, src_hbm, dst_hbm, out_hbm, sidx, didx, rows, acc, sem):
        c = lax.axis_index("c")
        s = lax.axis_index("s")
        w = c * NUM_SUBCORES + s
        _subcore_row_copy(lambda o, m: hs_hbm.at[pl.ds(o, m)],
                          lambda o, m: acc.at[pl.ds(o, m)], s, n_nodes)
        plsc.subcore_barrier()

        def body(j, carry):
            base = w * epw + j * EDGE_CHUNK
            pltpu.sync_copy(src_hbm.at[pl.ds(base, EDGE_CHUNK)], sidx)
            pltpu.sync_copy(dst_hbm.at[pl.ds(base, EDGE_CHUNK)], didx)
            pltpu.async_copy(hs_hbm.at[sidx], rows, sem).wait()
            pltpu.sync_copy(rows, acc.at[didx], add=True)
            return carry

        lax.fori_loop(0, nchunk, body, 0)
        plsc.subcore_barrier()
        _subcore_row_copy(lambda o, m: acc.at[pl.ds(o, m)],
                          lambda o, m: out_hbm.at[c, pl.ds(o, m)], s, n_nodes)

    return agg_kernel


ROW_BLK = 1000


def _tc1_body(dp_ref, x_ref, w1_ref, dis_ref, h1s_ref):
    deg = dp_ref[0] + dp_ref[1] + 1.0
    dis = lax.rsqrt(deg)
    dis_ref[...] = dis
    h = jnp.dot(x_ref[...], w1_ref[...], preferred_element_type=jnp.float32)
    h1s_ref[...] = h * dis[:, :1]


def _tc2_body(p_ref, h1s_ref, dis_ref, b1_ref, w2_ref, h2s_ref):
    dis = dis_ref[...][:, :1]
    agg = p_ref[0] + p_ref[1] - h1s_ref[...]
    a1 = jnp.maximum(agg * dis + b1_ref[...], 0.0)
    h2 = jnp.dot(a1, w2_ref[...], preferred_element_type=jnp.float32)
    h2s_ref[...] = h2 * dis


def _tc3_body(q_ref, h2s_ref, dis_ref, b2_ref, batch_ref, wout_ref, bout_ref,
              out_ref, sums, cnts, *, nblk, n_graphs):
    i = pl.program_id(0)

    @pl.when(i == 0)
    def _init():
        sums[...] = jnp.zeros_like(sums)
        cnts[...] = jnp.zeros_like(cnts)

    dis = dis_ref[...][:, :1]
    agg = q_ref[0] + q_ref[1] - h2s_ref[...]
    a2 = jnp.maximum(agg * dis + b2_ref[...], 0.0)

    seg = batch_ref[0, 0, :]
    iota = lax.broadcasted_iota(jnp.int32, (a2.shape[0], n_graphs), 1)
    onehot = (seg[:, None] == iota).astype(jnp.float32)
    dims = (((0,), (0,)), ((), ()))
    sums[...] += lax.dot_general(onehot, a2, dims,
                                 preferred_element_type=jnp.float32)
    cnts[...] += lax.dot_general(onehot, jnp.ones_like(a2), dims,
                                 preferred_element_type=jnp.float32)

    @pl.when(i == nblk - 1)
    def _fin():
        g = sums[...] / jnp.maximum(cnts[...], 1.0)
        logits = (jnp.dot(g, wout_ref[...], preferred_element_type=jnp.float32)
                  + bout_ref[...])
        m = jnp.max(logits, axis=1, keepdims=True)
        z = logits - m
        lse = jnp.log(jnp.sum(jnp.exp(z), axis=1, keepdims=True))
        out_ref[...] = z - lse


def kernel(x, edge_index, batch, W1, b1, W2, b2, Wout, bout):
    n, f_in = x.shape
    h = W1.shape[1]
    g_graphs = 64
    c_out = Wout.shape[1]
    e = edge_index.shape[1]
    src = edge_index[0]
    dst = edge_index[1]

    nblk = n // ROW_BLK

    deg_kernel = _make_degree_kernel(n, e)
    zeros16 = jnp.zeros((n, 16), jnp.float32)
    ones16 = jnp.ones((EDGE_CHUNK, 16), jnp.float32)
    degp = deg_kernel(dst, zeros16, ones16)

    dis, h1s = pl.pallas_call(
        _tc1_body,
        grid=(nblk,),
        in_specs=[
            pl.BlockSpec((NUM_CORES, ROW_BLK, 16), lambda i: (0, i, 0)),
            pl.BlockSpec((ROW_BLK, f_in), lambda i: (i, 0)),
            pl.BlockSpec((f_in, h), lambda i: (0, 0)),
        ],
        out_specs=[
            pl.BlockSpec((ROW_BLK, 16), lambda i: (i, 0)),
            pl.BlockSpec((ROW_BLK, h), lambda i: (i, 0)),
        ],
        out_shape=[
            jax.ShapeDtypeStruct((n, 16), jnp.float32),
            jax.ShapeDtypeStruct((n, h), jnp.float32),
        ],
    )(degp, x, W1)

    agg_kernel = _make_agg_kernel(n, e, h)
    p = agg_kernel(h1s, src, dst)

    h2s = pl.pallas_call(
        _tc2_body,
        grid=(nblk,),
        in_specs=[
            pl.BlockSpec((NUM_CORES, ROW_BLK, h), lambda i: (0, i, 0)),
            pl.BlockSpec((ROW_BLK, h), lambda i: (i, 0)),
            pl.BlockSpec((ROW_BLK, 16), lambda i: (i, 0)),
            pl.BlockSpec((1, h), lambda i: (0, 0)),
            pl.BlockSpec((h, h), lambda i: (0, 0)),
        ],
        out_specs=pl.BlockSpec((ROW_BLK, h), lambda i: (i, 0)),
        out_shape=jax.ShapeDtypeStruct((n, h), jnp.float32),
    )(p, h1s, dis, b1.reshape(1, h), W2)

    q = agg_kernel(h2s, src, dst)

    batch3d = batch.reshape(nblk, 1, ROW_BLK)
    out = pl.pallas_call(
        functools.partial(_tc3_body, nblk=nblk, n_graphs=g_graphs),
        grid=(nblk,),
        in_specs=[
            pl.BlockSpec((NUM_CORES, ROW_BLK, h), lambda i: (0, i, 0)),
            pl.BlockSpec((ROW_BLK, h), lambda i: (i, 0)),
            pl.BlockSpec((ROW_BLK, 16), lambda i: (i, 0)),
            pl.BlockSpec((1, h), lambda i: (0, 0)),
            pl.BlockSpec((1, 1, ROW_BLK), lambda i: (i, 0, 0)),
            pl.BlockSpec((h, c_out), lambda i: (0, 0)),
            pl.BlockSpec((1, c_out), lambda i: (0, 0)),
        ],
        out_specs=pl.BlockSpec((g_graphs, c_out), lambda i: (0, 0)),
        out_shape=jax.ShapeDtypeStruct((g_graphs, c_out), jnp.float32),
        scratch_shapes=[
            pltpu.VMEM((g_graphs, h), jnp.float32),
            pltpu.VMEM((g_graphs, h), jnp.float32),
        ],
    )(q, h2s, dis, b2.reshape(1, h), batch3d, Wout, bout.reshape(1, c_out))

    return out


# speedup vs baseline: 28.2898x; 2.1841x over previous
import functools

import jax
import jax.numpy as jnp
from jax import lax
from jax.experimental import pallas as pl
from jax.experimental.pallas import tpu as pltpu
from jax.experimental.pallas import tpu_sc as plsc

NUM_CORES = 2
NUM_SUBCORES = 16
NUM_TILES = NUM_CORES * NUM_SUBCORES
EDGE_CHUNK = 80


def _sc_mesh():
    return plsc.VectorSubcoreMesh(
        core_axis_name="c", subcore_axis_name="s",
        num_cores=NUM_CORES, num_subcores=NUM_SUBCORES)


def _row_partition(n_nodes):
    main = (n_nodes // (NUM_SUBCORES * 8)) * 8
    tail = n_nodes - NUM_SUBCORES * main
    return main, tail


def _subcore_row_copy(src_at, dst_at, s, n_nodes):
    main, tail = _row_partition(n_nodes)
    pltpu.sync_copy(src_at(s * main, main), dst_at(s * main, main))
    if tail:
        @pl.when(s == NUM_SUBCORES - 1)
        def _tail():
            pltpu.sync_copy(src_at(NUM_SUBCORES * main, tail),
                            dst_at(NUM_SUBCORES * main, tail))


def _make_degree_kernel(n_nodes, n_edges):
    epw = n_edges // NUM_TILES
    nchunk = epw // EDGE_CHUNK

    @functools.partial(
        pl.kernel,
        out_type=jax.ShapeDtypeStruct((NUM_CORES, n_nodes, 16), jnp.float32),
        mesh=_sc_mesh(),
        scratch_types=[
            pltpu.VMEM((EDGE_CHUNK,), jnp.int32),
            pltpu.VMEM((EDGE_CHUNK, 16), jnp.float32),
            pltpu.VMEM_SHARED((n_nodes, 16), jnp.float32),
        ],
    )
    def deg_kernel(dst_hbm, zeros_hbm, ones_hbm, out_hbm, didx, ones_v, acc):
        c = lax.axis_index("c")
        s = lax.axis_index("s")
        w = c * NUM_SUBCORES + s
        _subcore_row_copy(lambda o, m: zeros_hbm.at[pl.ds(o, m)],
                          lambda o, m: acc.at[pl.ds(o, m)], s, n_nodes)
        pltpu.sync_copy(ones_hbm, ones_v)
        plsc.subcore_barrier()

        def body(j, carry):
            pltpu.sync_copy(dst_hbm.at[w, j], didx)
            pltpu.sync_copy(ones_v, acc.at[didx], add=True)
            return carry

        lax.fori_loop(0, nchunk, body, 0)
        plsc.subcore_barrier()
        _subcore_row_copy(lambda o, m: acc.at[pl.ds(o, m)],
                          lambda o, m: out_hbm.at[c, pl.ds(o, m)], s, n_nodes)

    return deg_kernel


NBUF = 3


def _make_agg_kernel(n_nodes, n_edges, hdim):
    epw = n_edges // NUM_TILES
    nchunk = epw // EDGE_CHUNK

    @functools.partial(
        pl.kernel,
        out_type=jax.ShapeDtypeStruct((NUM_CORES, n_nodes, hdim), jnp.float32),
        mesh=_sc_mesh(),
        scratch_types=[
            pltpu.VMEM((nchunk, EDGE_CHUNK), jnp.int32),
        ] + [pltpu.VMEM((EDGE_CHUNK,), jnp.int32)] * NBUF + [
            pltpu.VMEM((NBUF, EDGE_CHUNK, hdim), jnp.float32),
            pltpu.VMEM_SHARED((n_nodes, hdim), jnp.float32),
        ] + [pltpu.SemaphoreType.DMA] * (2 * NBUF),
    )
    def agg_kernel(hs_hbm, src_hbm, dst_hbm, out_hbm, sidx, *rest):
        didxs = rest[:NBUF]
        bufs = rest[NBUF]
        acc = rest[NBUF + 1]
        sems = rest[NBUF + 2:]
        gsems = sems[:NBUF]
        dsems = sems[NBUF:]
        c = lax.axis_index("c")
        s = lax.axis_index("s")
        w = c * NUM_SUBCORES + s
        pltpu.sync_copy(src_hbm.at[w], sidx)
        _subcore_row_copy(lambda o, m: hs_hbm.at[pl.ds(o, m)],
                          lambda o, m: acc.at[pl.ds(o, m)], s, n_nodes)
        plsc.subcore_barrier()

        def fire(chunk, b):
            pltpu.async_copy(dst_hbm.at[w, chunk], didxs[b], dsems[b])
            pltpu.async_copy(hs_hbm.at[sidx.at[chunk]], bufs.at[b], gsems[b])

        def drain(b):
            pltpu.make_async_copy(dst_hbm.at[0, 0], didxs[b], dsems[b]).wait()
            pltpu.make_async_copy(hs_hbm.at[sidx.at[0]], bufs.at[b],
                                  gsems[b]).wait()

        for b in range(NBUF):
            fire(b, b)

        def body(j, carry):
            b = lax.rem(j, NBUF)
            for bb in range(NBUF):
                @pl.when(b == bb)
                def _do():
                    drain(bb)
                    pltpu.sync_copy(bufs.at[bb], acc.at[didxs[bb]], add=True)

                    @pl.when(j < nchunk - NBUF)
                    def _refill():
                        fire(j + NBUF, bb)
            return carry

        lax.fori_loop(0, nchunk, body, 0)
        plsc.subcore_barrier()
        _subcore_row_copy(lambda o, m: acc.at[pl.ds(o, m)],
                          lambda o, m: out_hbm.at[c, pl.ds(o, m)], s, n_nodes)

    return agg_kernel


ROW_BLK = 1000


def _tc1_body(dp_ref, x_ref, w1_ref, dis_ref, h1s_ref):
    deg = dp_ref[0] + dp_ref[1] + 1.0
    dis = lax.rsqrt(deg)
    dis_ref[...] = dis
    h = jnp.dot(x_ref[...], w1_ref[...], preferred_element_type=jnp.float32)
    h1s_ref[...] = h * dis[:, :1]


def _tc2_body(p_ref, h1s_ref, dis_ref, b1_ref, w2_ref, h2s_ref):
    dis = dis_ref[...][:, :1]
    agg = p_ref[0] + p_ref[1] - h1s_ref[...]
    a1 = jnp.maximum(agg * dis + b1_ref[...], 0.0)
    h2 = jnp.dot(a1, w2_ref[...], preferred_element_type=jnp.float32)
    h2s_ref[...] = h2 * dis


def _tc3_body(q_ref, h2s_ref, dis_ref, b2_ref, batch_ref, wout_ref, bout_ref,
              out_ref, sums, cnts, *, nblk, n_graphs):
    i = pl.program_id(0)

    @pl.when(i == 0)
    def _init():
        sums[...] = jnp.zeros_like(sums)
        cnts[...] = jnp.zeros_like(cnts)

    dis = dis_ref[...][:, :1]
    agg = q_ref[0] + q_ref[1] - h2s_ref[...]
    a2 = jnp.maximum(agg * dis + b2_ref[...], 0.0)

    seg = batch_ref[0, 0, :]
    iota = lax.broadcasted_iota(jnp.int32, (a2.shape[0], n_graphs), 1)
    onehot = (seg[:, None] == iota).astype(jnp.float32)
    dims = (((0,), (0,)), ((), ()))
    sums[...] += lax.dot_general(onehot, a2, dims,
                                 preferred_element_type=jnp.float32)
    cnts[...] += lax.dot_general(onehot, jnp.ones_like(a2), dims,
                                 preferred_element_type=jnp.float32)

    @pl.when(i == nblk - 1)
    def _fin():
        g = sums[...] / jnp.maximum(cnts[...], 1.0)
        logits = (jnp.dot(g, wout_ref[...], preferred_element_type=jnp.float32)
                  + bout_ref[...])
        m = jnp.max(logits, axis=1, keepdims=True)
        z = logits - m
        lse = jnp.log(jnp.sum(jnp.exp(z), axis=1, keepdims=True))
        out_ref[...] = z - lse


def kernel(x, edge_index, batch, W1, b1, W2, b2, Wout, bout):
    n, f_in = x.shape
    h = W1.shape[1]
    g_graphs = 64
    c_out = Wout.shape[1]
    e = edge_index.shape[1]
    nchunk = e // (NUM_TILES * EDGE_CHUNK)
    src3 = edge_index[0].reshape(NUM_TILES, nchunk, EDGE_CHUNK)
    dst3 = edge_index[1].reshape(NUM_TILES, nchunk, EDGE_CHUNK)

    nblk = n // ROW_BLK

    deg_kernel = _make_degree_kernel(n, e)
    zeros16 = jnp.zeros((n, 16), jnp.float32)
    ones16 = jnp.ones((EDGE_CHUNK, 16), jnp.float32)
    degp = deg_kernel(dst3, zeros16, ones16)

    dis, h1s = pl.pallas_call(
        _tc1_body,
        grid=(nblk,),
        in_specs=[
            pl.BlockSpec((NUM_CORES, ROW_BLK, 16), lambda i: (0, i, 0)),
            pl.BlockSpec((ROW_BLK, f_in), lambda i: (i, 0)),
            pl.BlockSpec((f_in, h), lambda i: (0, 0)),
        ],
        out_specs=[
            pl.BlockSpec((ROW_BLK, 16), lambda i: (i, 0)),
            pl.BlockSpec((ROW_BLK, h), lambda i: (i, 0)),
        ],
        out_shape=[
            jax.ShapeDtypeStruct((n, 16), jnp.float32),
            jax.ShapeDtypeStruct((n, h), jnp.float32),
        ],
    )(degp, x, W1)

    agg_kernel = _make_agg_kernel(n, e, h)
    p = agg_kernel(h1s, src3, dst3)

    h2s = pl.pallas_call(
        _tc2_body,
        grid=(nblk,),
        in_specs=[
            pl.BlockSpec((NUM_CORES, ROW_BLK, h), lambda i: (0, i, 0)),
            pl.BlockSpec((ROW_BLK, h), lambda i: (i, 0)),
            pl.BlockSpec((ROW_BLK, 16), lambda i: (i, 0)),
            pl.BlockSpec((1, h), lambda i: (0, 0)),
            pl.BlockSpec((h, h), lambda i: (0, 0)),
        ],
        out_specs=pl.BlockSpec((ROW_BLK, h), lambda i: (i, 0)),
        out_shape=jax.ShapeDtypeStruct((n, h), jnp.float32),
    )(p, h1s, dis, b1.reshape(1, h), W2)

    q = agg_kernel(h2s, src3, dst3)

    batch3d = batch.reshape(nblk, 1, ROW_BLK)
    out = pl.pallas_call(
        functools.partial(_tc3_body, nblk=nblk, n_graphs=g_graphs),
        grid=(nblk,),
        in_specs=[
            pl.BlockSpec((NUM_CORES, ROW_BLK, h), lambda i: (0, i, 0)),
            pl.BlockSpec((ROW_BLK, h), lambda i: (i, 0)),
            pl.BlockSpec((ROW_BLK, 16), lambda i: (i, 0)),
            pl.BlockSpec((1, h), lambda i: (0, 0)),
            pl.BlockSpec((1, 1, ROW_BLK), lambda i: (i, 0, 0)),
            pl.BlockSpec((h, c_out), lambda i: (0, 0)),
            pl.BlockSpec((1, c_out), lambda i: (0, 0)),
        ],
        out_specs=pl.BlockSpec((g_graphs, c_out), lambda i: (0, 0)),
        out_shape=jax.ShapeDtypeStruct((g_graphs, c_out), jnp.float32),
        scratch_shapes=[
            pltpu.VMEM((g_graphs, h), jnp.float32),
            pltpu.VMEM((g_graphs, h), jnp.float32),
        ],
    )(q, h2s, dis, b2.reshape(1, h), batch3d, Wout, bout.reshape(1, c_out))

    return out
